# baseline (speedup 1.0000x reference)
import functools
import math

import numpy as np
import jax
import jax.numpy as jnp
from jax import lax
from jax.experimental import pallas as pl
from jax.experimental.pallas import tpu as pltpu
from jax.experimental.pallas import tpu_sc as plsc

_B = 4096
_S = 200
_D = 64
_NV = 1000000

_INFO = plsc.get_sparse_core_info()
_NC = _INFO.num_cores
_NS = _INFO.num_subcores
_NW = _NC * _NS
_TOTAL = _B * _S
_PER_W = _TOTAL // _NW
_C = 128
_NCHUNK = _PER_W // _C
_NBUF = 2
_NGROUP = _NCHUNK // _NBUF
_LANES = 16

assert _PER_W % _C == 0 and _NCHUNK % _NBUF == 0 and _C % 8 == 0


def _make_pe_ext():
    position = np.arange(0, _S, dtype=np.float64)[:, None]
    div_term = np.exp(
        np.arange(0, _D, 2, dtype=np.float64) * -(math.log(10000.0) / _D))
    pe = np.zeros((_S, _D), dtype=np.float64)
    pe[:, 0::2] = np.sin(position * div_term)
    pe[:, 1::2] = np.cos(position * div_term)
    reps = -(-(_S + _C) // _S)
    pe_ext = np.concatenate([pe] * reps, axis=0)[: _S + _C]
    return jnp.asarray(pe_ext, dtype=jnp.float32)


_mesh = plsc.VectorSubcoreMesh(core_axis_name="c", subcore_axis_name="s")


@functools.partial(
    pl.kernel,
    mesh=_mesh,
    out_type=jax.ShapeDtypeStruct((_TOTAL, _D), jnp.float32),
    scratch_types=[
        pltpu.VMEM((_NBUF, _C), jnp.int32),
        pltpu.VMEM((_S + _C, _D), jnp.float32),
        pltpu.VMEM((_NBUF, _C, _D), jnp.float32),
        pltpu.VMEM((_NBUF, _C, _D), jnp.float32),
        pltpu.SemaphoreType.DMA((_NBUF,)),
        pltpu.SemaphoreType.DMA((_NBUF,)),
        pltpu.SemaphoreType.DMA((_NBUF,)),
    ],
    compiler_params=pltpu.CompilerParams(use_tc_tiling_on_sc=True),
)
def _gather_pe(table, idxf, pef, out, idx_s, pe_v, gbuf, sbuf,
               isem, gsem, ssem):
    wid = lax.axis_index("s") * _NC + lax.axis_index("c")
    base = wid * _PER_W
    pltpu.sync_copy(pef, pe_v)

    def idx_start(j, k):
        pltpu.async_copy(
            idxf.at[pl.ds(base + j * _C, _C)], idx_s.at[k], isem.at[k])

    def idx_wait(j, k):
        pltpu.make_async_copy(
            idxf.at[pl.ds(base + j * _C, _C)], idx_s.at[k], isem.at[k]).wait()

    def gather_start(k):
        def grp_dma(gi, carry):
            vec = idx_s[k, pl.ds(gi * _LANES, _LANES)]
            for e in range(_LANES):
                s = vec[e]
                pltpu.async_copy(
                    table.at[pl.ds(s, 1)],
                    gbuf.at[k, pl.ds(gi * _LANES + e, 1)], gsem.at[k])
            return carry

        lax.fori_loop(0, _C // _LANES, grp_dma, 0, unroll=False)

    def gather_wait(k):
        pltpu.make_async_copy(
            table.at[pl.ds(0, _C)], gbuf.at[k], gsem.at[k]).wait()

    def scatter_start(j, k):
        pltpu.async_copy(
            sbuf.at[k], out.at[pl.ds(base + j * _C, _C)], ssem.at[k])

    def scatter_wait(j, k):
        pltpu.make_async_copy(
            sbuf.at[k], out.at[pl.ds(base + j * _C, _C)], ssem.at[k]).wait()

    for k in range(_NBUF):
        idx_start(k, k)
    for k in range(_NBUF):
        idx_wait(k, k)
        gather_start(k)

    def group_body(g, carry):
        for k in range(_NBUF):
            j = g * _NBUF + k
            gather_wait(k)

            @pl.when(g < _NGROUP - 1)
            def _():
                idx_start(j + _NBUF, k)

            @pl.when(g > 0)
            def _():
                scatter_wait(j - _NBUF, k)

            pos0 = lax.rem(j * _C, _S)

            def row_body(r, carry2):
                for d in range(_D // _LANES):
                    sl = pl.ds(d * _LANES, _LANES)
                    sbuf[k, r, sl] = (
                        gbuf[k, r, sl] * 8.0 + pe_v[pos0 + r, sl])
                return carry2

            lax.fori_loop(0, _C, row_body, 0, unroll=4)
            scatter_start(j, k)

            @pl.when(g < _NGROUP - 1)
            def _():
                idx_wait(j + _NBUF, k)
                gather_start(k)
        return carry

    lax.fori_loop(0, _NGROUP, group_body, 0, unroll=False)

    for k in range(_NBUF):
        scatter_wait(_NCHUNK - _NBUF + k, k)


def kernel(x, weight):
    pe_ext = _make_pe_ext()
    xf = x.reshape(_TOTAL).astype(jnp.int32)
    out = _gather_pe(weight, xf, pe_ext)
    return out.reshape(_B, _S, _D)


# speedup vs baseline: 1.2002x; 1.2002x over previous
import functools
import math

import numpy as np
import jax
import jax.numpy as jnp
from jax import lax
from jax.experimental import pallas as pl
from jax.experimental.pallas import tpu as pltpu
from jax.experimental.pallas import tpu_sc as plsc

_B = 4096
_S = 200
_D = 64
_NV = 1000000

_INFO = plsc.get_sparse_core_info()
_NC = _INFO.num_cores
_NS = _INFO.num_subcores
_NW = _NC * _NS
_TOTAL = _B * _S
_BPW = _B // _NW
_NBUF = 2
_NGROUP = _BPW // _NBUF
_LANES = 16

assert _B % _NW == 0 and _BPW % _NBUF == 0


def _make_pe():
    position = np.arange(0, _S, dtype=np.float64)[:, None]
    div_term = np.exp(
        np.arange(0, _D, 2, dtype=np.float64) * -(math.log(10000.0) / _D))
    pe = np.zeros((_S, _D), dtype=np.float64)
    pe[:, 0::2] = np.sin(position * div_term)
    pe[:, 1::2] = np.cos(position * div_term)
    return jnp.asarray(pe, dtype=jnp.float32)


_mesh = plsc.VectorSubcoreMesh(core_axis_name="c", subcore_axis_name="s")


@functools.partial(
    pl.kernel,
    mesh=_mesh,
    out_type=jax.ShapeDtypeStruct((_TOTAL, _D), jnp.float32),
    scratch_types=[
        pltpu.VMEM((_NBUF, 1, _S), jnp.int32),
        pltpu.VMEM((_S, _D), jnp.float32),
        pltpu.VMEM((_NBUF, _S, _D), jnp.float32),
        pltpu.VMEM((_NBUF, _S, _D), jnp.float32),
        pltpu.SemaphoreType.DMA((_NBUF,)),
        pltpu.SemaphoreType.DMA((_NBUF,)),
        pltpu.SemaphoreType.DMA((_NBUF,)),
    ],
    compiler_params=pltpu.CompilerParams(use_tc_tiling_on_sc=True),
)
def _gather_pe(table, xmat, pef, out, idx_s, pe_v, gbuf, sbuf,
               isem, gsem, ssem):
    wid = lax.axis_index("s") * _NC + lax.axis_index("c")
    b0 = wid * _BPW
    pltpu.sync_copy(pef, pe_v)

    def idx_start(j, k):
        pltpu.async_copy(
            xmat.at[pl.ds(b0 + j, 1)], idx_s.at[k], isem.at[k])

    def idx_wait(j, k):
        pltpu.make_async_copy(
            xmat.at[pl.ds(b0 + j, 1)], idx_s.at[k], isem.at[k]).wait()

    def gather_start(k):
        def grp_dma(gi, carry):
            vec = idx_s[k, 0, pl.ds(gi * _LANES, _LANES)]
            for e in range(_LANES):
                r = gi * _LANES + e
                s = vec[e]
                pltpu.async_copy(
                    table.at[pl.ds(s, 1)], gbuf.at[k, pl.ds(r, 1)],
                    gsem.at[k])
            return carry

        lax.fori_loop(0, _S // _LANES, grp_dma, 0, unroll=False)
        vec = idx_s[k, 0, pl.ds(_S - _LANES, _LANES)]
        for e in range(_S % _LANES):
            r = (_S // _LANES) * _LANES + e
            s = vec[(_LANES - _S % _LANES) + e]
            pltpu.async_copy(
                table.at[pl.ds(s, 1)], gbuf.at[k, pl.ds(r, 1)], gsem.at[k])

    def gather_wait(k):
        pltpu.make_async_copy(
            table.at[pl.ds(0, _S)], gbuf.at[k], gsem.at[k]).wait()

    def scatter_start(j, k):
        pltpu.async_copy(
            sbuf.at[k], out.at[pl.ds((b0 + j) * _S, _S)], ssem.at[k])

    def scatter_wait(j, k):
        pltpu.make_async_copy(
            sbuf.at[k], out.at[pl.ds((b0 + j) * _S, _S)], ssem.at[k]).wait()

    for k in range(_NBUF):
        idx_start(k, k)
    for k in range(_NBUF):
        idx_wait(k, k)
        gather_start(k)

    def group_body(g, carry):
        for k in range(_NBUF):
            j = g * _NBUF + k
            gather_wait(k)

            @pl.when(g < _NGROUP - 1)
            def _():
                idx_start(j + _NBUF, k)

            @pl.when(g > 0)
            def _():
                scatter_wait(j - _NBUF, k)

            def row_body(r, carry2):
                for d in range(_D // _LANES):
                    sl = pl.ds(d * _LANES, _LANES)
                    sbuf[k, r, sl] = gbuf[k, r, sl] * 8.0 + pe_v[r, sl]
                return carry2

            lax.fori_loop(0, _S, row_body, 0, unroll=4)
            scatter_start(j, k)

            @pl.when(g < _NGROUP - 1)
            def _():
                idx_wait(j + _NBUF, k)
                gather_start(k)
        return carry

    lax.fori_loop(0, _NGROUP, group_body, 0, unroll=False)

    for k in range(_NBUF):
        scatter_wait(_BPW - _NBUF + k, k)


def kernel(x, weight):
    pe = _make_pe()
    out = _gather_pe(weight, x.astype(jnp.int32), pe)
    return out.reshape(_B, _S, _D)
